# baseline (speedup 1.0000x reference)
import jax
import jax.numpy as jnp
import numpy as np
from jax import lax
from jax.experimental import pallas as pl
from jax.experimental.pallas import tpu as pltpu
from jax.experimental.pallas import tpu_sc as plsc

_B, _N, _F = 8, 5000, 85
_NDET = 100
_NC, _NS = 1, 16
_NW = _NC * _NS
_RPT = 128 // _NW
_PAD_ROWS = _NW * _RPT
_LANES = 16
_BAND_LO, _BAND_ROWS = 96, 104

_BATCHES = np.array(
    [0, 0, 0, 0, 0, 0, 0, 0, 0, 0, 0, 1, 1, 1, 1, 1, 1, 1, 1, 1, 1, 1, 1, 1,
     2, 2, 2, 2, 2, 2, 2, 2, 2, 2, 2, 2, 2, 2, 2, 2, 2, 2, 2, 2, 2, 2, 2,
     3, 3, 3, 3, 3, 3, 3, 3, 3, 3, 4, 4, 4, 4, 4, 4, 4, 4, 4, 4, 4, 4,
     5, 5, 5, 5, 5, 5, 5, 5, 5, 5, 5, 6, 6, 6, 6, 6, 6, 6, 6, 6,
     7, 7, 7, 7, 7, 7, 7, 7, 7, 7, 7], dtype=np.int64)


def _aux_const():
    aux = np.zeros((_NW, 2 * _RPT, 128), np.float32)
    for w in range(_NW):
        for j in range(_RPT):
            r = w * _RPT + j
            if r < _NDET:
                b = int(_BATCHES[r])
                aux[w, j, :] = b * _BAND_ROWS + (100 + r - _BAND_LO)
                aux[w, _RPT + j, :] = b
    return aux


_AUX_NP = _aux_const()
_CHUNK_BASES = (4, 20, 36, 52, 68, _F - _LANES)


def _sc_body(x, aux, cmp, out, buf, auxb, cmb, obuf, sem):
    wid = lax.axis_index("s") * _NC + lax.axis_index("c")
    copies = [pltpu.async_copy(x.at[b, pl.ds(_BAND_LO, _BAND_ROWS), :],
                               buf.at[pl.ds(b * _BAND_ROWS, _BAND_ROWS), :], sem)
              for b in range(_B)]
    copies.append(pltpu.async_copy(aux.at[wid], auxb, sem))
    copies.append(pltpu.async_copy(cmp, cmb, sem))
    for c in copies:
        c.wait()

    lane = lax.iota(jnp.int32, _LANES)
    zero = jnp.zeros((_LANES,), jnp.float32)
    cshift = jnp.clip(lane - 1, 0, 3)
    boxmask = jnp.where((lane >= 1) & (lane <= 4), 1.0, 0.0).astype(jnp.float32)
    cmrows = [plsc.load_gather(cmb, [jnp.full((_LANES,), k, jnp.int32), cshift])
              * boxmask for k in range(4)]

    def per_row(j, carry):
        rowv = auxb[j, pl.ds(0, _LANES)].astype(jnp.int32)
        bf = auxb[_RPT + j, pl.ds(0, _LANES)]
        acc = zero
        for k in range(4):
            xk = plsc.load_gather(buf, [rowv, jnp.full((_LANES,), k, jnp.int32)])
            acc = acc + xk * cmrows[k]
        chunks = [plsc.load_gather(buf, [rowv, c + lane]) for c in _CHUNK_BASES]
        m = chunks[0]
        for v in chunks[1:]:
            m = jnp.maximum(m, v)
        mx = jnp.broadcast_to(jnp.max(m), (_LANES,))
        big = jnp.full((_LANES,), 10000, jnp.int32)
        am = big
        for c, v in zip(_CHUNK_BASES, chunks):
            am = jnp.minimum(am, jnp.where(v == mx, lane + (c - 4), big))
        amf = jnp.broadcast_to(jnp.min(am), (_LANES,)).astype(jnp.float32)
        row = acc + jnp.where(lane == 0, bf, zero)
        row = row + jnp.where(lane == 5, amf, zero)
        row = row + jnp.where(lane == 6, mx, zero)
        obuf[j, pl.ds(0, _LANES)] = row
        return carry

    lax.fori_loop(0, _RPT, per_row, 0, unroll=False)
    pltpu.sync_copy(obuf, out.at[pl.ds(wid * _RPT, _RPT)])


@jax.jit
def kernel(x, convert_matrix):
    aux = jnp.asarray(_AUX_NP)
    cmp = jnp.pad(convert_matrix, ((0, 4), (0, 124)))
    run = pl.kernel(
        _sc_body,
        out_type=jax.ShapeDtypeStruct((_PAD_ROWS, 128), jnp.float32),
        mesh=plsc.VectorSubcoreMesh(
            core_axis_name="c", subcore_axis_name="s",
            num_cores=_NC, num_subcores=_NS),
        scratch_types=[
            pltpu.VMEM((_B * _BAND_ROWS, _F), jnp.float32),
            pltpu.VMEM((2 * _RPT, 128), jnp.float32),
            pltpu.VMEM((8, 128), jnp.float32),
            pltpu.VMEM((_RPT, 128), jnp.float32),
            pltpu.SemaphoreType.DMA,
        ],
        compiler_params=pltpu.CompilerParams(
            needs_layout_passes=False, skip_device_barrier=True,
            use_tc_tiling_on_sc=True),
    )
    padded = run(x, aux, cmp)
    return padded[:_NDET, :7]


# speedup vs baseline: 1.8710x; 1.8710x over previous
import jax
import jax.numpy as jnp
import numpy as np
from jax import lax
from jax.experimental import pallas as pl
from jax.experimental.pallas import tpu as pltpu
from jax.experimental.pallas import tpu_sc as plsc

_B, _N, _F = 8, 5000, 85
_NDET = 100
_NC, _NS = 1, 16
_NW = _NC * _NS
_RPT = 128 // _NW
_PAD_ROWS = _NW * _RPT
_LANES = 16
_BAND_LO, _BAND_ROWS = 96, 104

_BATCHES = np.array(
    [0, 0, 0, 0, 0, 0, 0, 0, 0, 0, 0, 1, 1, 1, 1, 1, 1, 1, 1, 1, 1, 1, 1, 1,
     2, 2, 2, 2, 2, 2, 2, 2, 2, 2, 2, 2, 2, 2, 2, 2, 2, 2, 2, 2, 2, 2, 2,
     3, 3, 3, 3, 3, 3, 3, 3, 3, 3, 4, 4, 4, 4, 4, 4, 4, 4, 4, 4, 4, 4,
     5, 5, 5, 5, 5, 5, 5, 5, 5, 5, 5, 6, 6, 6, 6, 6, 6, 6, 6, 6,
     7, 7, 7, 7, 7, 7, 7, 7, 7, 7, 7], dtype=np.int64)


def _runs():
    runs = []
    rowof = np.zeros((_PAD_ROWS,), np.int64)
    g = 0
    i = 0
    while i < _NDET:
        j = i
        while j < _NDET and _BATCHES[j] == _BATCHES[i]:
            j += 1
        lo8 = i - (i % 8)
        n8 = -(-(j - lo8) // 8) * 8
        runs.append((int(_BATCHES[i]), int(lo8), int(n8), int(g)))
        rowof[i:j] = g + (np.arange(i, j) - lo8)
        g += n8
        i = j
    return runs, rowof, int(g)


_RUNS, _ROWOF, _BUF_ROWS = _runs()


def _aux_const():
    aux = np.zeros((_NW, 2 * _RPT, _LANES), np.float32)
    for w in range(_NW):
        for j in range(_RPT):
            r = w * _RPT + j
            if r < _NDET:
                aux[w, j, :] = _ROWOF[r]
                aux[w, _RPT + j, :] = int(_BATCHES[r])
    return aux


_AUX_NP = _aux_const()
_CHUNK_BASES = (4, 20, 36, 52, 68, _F - _LANES)


def _sc_body(y, aux, cmp, out, buf, auxb, cmb, obuf, sem):
    wid = lax.axis_index("s") * _NC + lax.axis_index("c")
    copies = [pltpu.async_copy(y.at[b, pl.ds(lo8, n8), :],
                               buf.at[pl.ds(g, n8), :], sem)
              for (b, lo8, n8, g) in _RUNS]
    copies.append(pltpu.async_copy(aux.at[wid], auxb, sem))
    copies.append(pltpu.async_copy(cmp, cmb, sem))
    for c in copies:
        c.wait()

    lane = lax.iota(jnp.int32, _LANES)
    zero = jnp.zeros((_LANES,), jnp.float32)
    cshift = jnp.clip(lane - 1, 0, 3)
    boxmask = jnp.where((lane >= 1) & (lane <= 4), 1.0, 0.0).astype(jnp.float32)
    cmrows = [plsc.load_gather(cmb, [jnp.full((_LANES,), k, jnp.int32), cshift])
              * boxmask for k in range(4)]

    def per_row(j, carry):
        rowv = auxb[j, :].astype(jnp.int32)
        bf = auxb[_RPT + j, :]
        acc = zero
        for k in range(4):
            xk = plsc.load_gather(buf, [rowv, jnp.full((_LANES,), k, jnp.int32)])
            acc = acc + xk * cmrows[k]
        chunks = [plsc.load_gather(buf, [rowv, c + lane]) for c in _CHUNK_BASES]
        m = chunks[0]
        for v in chunks[1:]:
            m = jnp.maximum(m, v)
        mx = jnp.broadcast_to(jnp.max(m), (_LANES,))
        big = jnp.full((_LANES,), 10000, jnp.int32)
        am = big
        for c, v in zip(_CHUNK_BASES, chunks):
            am = jnp.minimum(am, jnp.where(v == mx, lane + (c - 4), big))
        amf = jnp.broadcast_to(jnp.min(am), (_LANES,)).astype(jnp.float32)
        row = acc + jnp.where(lane == 0, bf, zero)
        row = row + jnp.where(lane == 5, amf, zero)
        row = row + jnp.where(lane == 6, mx, zero)
        obuf[j, :] = row
        return carry

    lax.fori_loop(0, _RPT, per_row, 0, unroll=False)
    pltpu.sync_copy(obuf, out.at[pl.ds(wid * _RPT, _RPT)])


@jax.jit
def kernel(x, convert_matrix):
    y = x[:, 100:204, :]
    aux = jnp.asarray(_AUX_NP)
    run = pl.kernel(
        _sc_body,
        out_type=jax.ShapeDtypeStruct((_PAD_ROWS, _LANES), jnp.float32),
        mesh=plsc.VectorSubcoreMesh(
            core_axis_name="c", subcore_axis_name="s",
            num_cores=_NC, num_subcores=_NS),
        scratch_types=[
            pltpu.VMEM((_BUF_ROWS, _F), jnp.float32),
            pltpu.VMEM((2 * _RPT, _LANES), jnp.float32),
            pltpu.VMEM((4, 4), jnp.float32),
            pltpu.VMEM((_RPT, _LANES), jnp.float32),
            pltpu.SemaphoreType.DMA,
        ],
        compiler_params=pltpu.CompilerParams(
            needs_layout_passes=False, skip_device_barrier=True),
    )
    padded = run(y, aux, convert_matrix)
    return padded[:_NDET, :7]


# speedup vs baseline: 1.9141x; 1.0231x over previous
import jax
import jax.numpy as jnp
import numpy as np
from jax import lax
from jax.experimental import pallas as pl
from jax.experimental.pallas import tpu as pltpu
from jax.experimental.pallas import tpu_sc as plsc

_B, _N, _F = 8, 5000, 85
_NDET = 100
_NC, _NS = 1, 16
_NW = _NC * _NS
_RPT = 128 // _NW
_PAD_ROWS = _NW * _RPT
_LANES = 16

_BATCHES = np.array(
    [0, 0, 0, 0, 0, 0, 0, 0, 0, 0, 0, 1, 1, 1, 1, 1, 1, 1, 1, 1, 1, 1, 1, 1,
     2, 2, 2, 2, 2, 2, 2, 2, 2, 2, 2, 2, 2, 2, 2, 2, 2, 2, 2, 2, 2, 2, 2,
     3, 3, 3, 3, 3, 3, 3, 3, 3, 3, 4, 4, 4, 4, 4, 4, 4, 4, 4, 4, 4, 4,
     5, 5, 5, 5, 5, 5, 5, 5, 5, 5, 5, 6, 6, 6, 6, 6, 6, 6, 6, 6,
     7, 7, 7, 7, 7, 7, 7, 7, 7, 7, 7], dtype=np.int64)


def _selection_tables():
    batches = _BATCHES
    runs = []
    offs = np.zeros((_PAD_ROWS,), np.int64)
    g = 0
    i = 0
    while i < _NDET:
        j = i
        while j < _NDET and batches[j] == batches[i]:
            j += 1
        b = int(batches[i])
        s = (b * _NDET + i) * _F
        e = (b * _NDET + j) * _F
        a = s - (s % 8)
        lp = -(-(e - a) // 8) * 8
        runs.append((int(a), int(g), int(lp)))
        offs[i:j] = g + (s - a) + (np.arange(j - i) * _F)
        g += lp
        i = j
    buf_words = g + 8
    aux = np.zeros((_NW, 2 * _RPT, _LANES), np.float32)
    bpad = np.zeros((_PAD_ROWS,), np.int64)
    bpad[:_NDET] = batches
    for w in range(_NW):
        for j in range(_RPT):
            r = w * _RPT + j
            aux[w, j, :] = offs[r]
            aux[w, _RPT + j, :] = bpad[r]
    return runs, aux, int(buf_words)


_RUNS, _AUX_NP, _BUF_WORDS = _selection_tables()
_CHUNK_BASES = (4, 20, 36, 52, 68, _F - _LANES)


def _sc_body(x1d, aux, cm, out, buf, auxb, cmb, obuf, sem):
    wid = lax.axis_index("s") * _NC + lax.axis_index("c")
    copies = [pltpu.async_copy(x1d.at[pl.ds(a, l)], buf.at[pl.ds(g, l)], sem)
              for (a, g, l) in _RUNS]
    copies.append(pltpu.async_copy(aux.at[wid], auxb, sem))
    copies.append(pltpu.async_copy(cm, cmb, sem))
    for c in copies:
        c.wait()

    lane = lax.iota(jnp.int32, _LANES)
    zero = jnp.zeros((_LANES,), jnp.float32)
    cshift = jnp.clip(lane - 1, 0, 3)
    boxmask = jnp.where((lane >= 1) & (lane <= 4), 1.0, 0.0).astype(jnp.float32)
    cmrows = [plsc.load_gather(cmb, [jnp.full((_LANES,), k, jnp.int32), cshift])
              * boxmask for k in range(4)]
    def per_row(j, carry):
        offv = auxb[j, :].astype(jnp.int32)
        bf = auxb[_RPT + j, :]
        acc = zero
        for k in range(4):
            xk = plsc.load_gather(buf, [offv + k])
            acc = acc + xk * cmrows[k]
        chunks = [plsc.load_gather(buf, [offv + (c + lane)]) for c in _CHUNK_BASES]
        m = chunks[0]
        for v in chunks[1:]:
            m = jnp.maximum(m, v)
        mx = jnp.broadcast_to(jnp.max(m), (_LANES,))
        big = jnp.full((_LANES,), 10000, jnp.int32)
        am = big
        for c, v in zip(_CHUNK_BASES, chunks):
            am = jnp.minimum(am, jnp.where(v == mx, lane + (c - 4), big))
        amf = jnp.broadcast_to(jnp.min(am), (_LANES,)).astype(jnp.float32)
        row = acc + jnp.where(lane == 0, bf, zero)
        row = row + jnp.where(lane == 5, amf, zero)
        row = row + jnp.where(lane == 6, mx, zero)
        obuf[j, :] = row
        return carry

    lax.fori_loop(0, _RPT, per_row, 0, unroll=False)
    pltpu.sync_copy(obuf, out.at[pl.ds(wid * _RPT, _RPT)])


@jax.jit
def kernel(x, convert_matrix):
    x1d = x[:, 100:200, :].reshape(-1)
    aux = jnp.asarray(_AUX_NP)
    run = pl.kernel(
        _sc_body,
        out_type=jax.ShapeDtypeStruct((_PAD_ROWS, _LANES), jnp.float32),
        mesh=plsc.VectorSubcoreMesh(
            core_axis_name="c", subcore_axis_name="s",
            num_cores=_NC, num_subcores=_NS),
        scratch_types=[
            pltpu.VMEM((_BUF_WORDS,), jnp.float32),
            pltpu.VMEM((2 * _RPT, _LANES), jnp.float32),
            pltpu.VMEM((4, 4), jnp.float32),
            pltpu.VMEM((_RPT, _LANES), jnp.float32),
            pltpu.SemaphoreType.DMA,
        ],
        compiler_params=pltpu.CompilerParams(
            needs_layout_passes=False, skip_device_barrier=True),
    )
    padded = run(x1d, aux, convert_matrix)
    return padded[:_NDET, :7]
